# baseline (speedup 1.0000x reference)
import math

import jax
import jax.numpy as jnp
from jax import lax
from jax.experimental import pallas as pl
from jax.experimental.pallas import tpu as pltpu
from jax.experimental.pallas import tpu_sc as plsc

_BS = 16
_FC = 128
_N = 16384
_MAP_HW = 101
_CELLS = _MAP_HW * _MAP_HW
_CELLS_PAD = 10240
_SENT = _CELLS
_EPS = -1e16
_MAP_SCALE = 0.1
_MAX_DEPTH = 10.0
_HFOV = math.radians(90.0)
_W = 512
_FX = _W / 2 * (1.0 / math.tan(_HFOV / 2))
_CX = _W / 2
_NC = 2
_NS = 16
_NW = _NC * _NS
_FPW = _FC // (_NW // _BS)


def _index_body(dsub_ref, lin_ref):
    z = dsub_ref[...] * _MAX_DEPTH
    valid = jnp.abs(z) > 0.8
    zf = jnp.round(-(z / _MAP_SCALE) + (_MAP_HW - 1))
    j = lax.broadcasted_iota(jnp.int32, (_BS, 128, 128), 2).astype(jnp.float32)
    x = j * 4.0 + 2.0
    xx = (x - _CX) / _FX
    xf = jnp.round((xx * z) / _MAP_SCALE + (_MAP_HW - 1) / 2)
    r0 = zf.astype(jnp.int32)
    c0 = xf.astype(jnp.int32)
    invalid = (
        (r0 >= _MAP_HW) | (c0 >= _MAP_HW) | (r0 < 0) | (c0 < 0)
        | jnp.logical_not(valid)
    )
    lin_ref[...] = jnp.where(invalid, _SENT, r0 * _MAP_HW + c0)


def _sc_scatter(feats_hbm, lin_hbm, out_hbm, lin_v, feats_v, map_v, acc_v):
    c = lax.axis_index("c")
    s = lax.axis_index("s")
    wid = s * _NC + c
    b = wid // 2
    fbase = (wid % 2) * _FPW
    pltpu.sync_copy(lin_hbm.at[b], lin_v)
    eps16 = jnp.full((16,), _EPS, jnp.float32)
    zero16 = jnp.zeros((16,), jnp.int32)
    one16 = jnp.ones((16,), jnp.int32)
    stride_iota = lax.iota(jnp.int32, 16) * (_N // 16)

    def pair(fi, carry):
        row = b * _FC + fbase + fi
        pltpu.sync_copy(feats_hbm.at[row], feats_v)
        acc_v[pl.ds(0, 16)] = zero16

        def init_step(g, cc):
            for k in range(8):
                map_v[pl.ds((g * 8 + k) * 16, 16)] = eps16
            return cc

        lax.fori_loop(0, _CELLS_PAD // 128, init_step, 0)

        def rmw_pass(g, cc):
            for k in range(8):
                pidx = stride_iota + g + k * 128
                idx = plsc.load_gather(lin_v, [pidx])
                val = plsc.load_gather(feats_v, [pidx])
                cur = plsc.load_gather(map_v, [idx])
                plsc.store_scatter(map_v, [idx], val, mask=val > cur)
            return cc

        lax.fori_loop(0, _N // 128, rmw_pass, 0)
        lax.fori_loop(0, _N // 128, rmw_pass, 0)

        def detect_pass(g, cc):
            bad = zero16
            for k in range(8):
                pidx = stride_iota + g + k * 128
                idx = plsc.load_gather(lin_v, [pidx])
                val = plsc.load_gather(feats_v, [pidx])
                cur = plsc.load_gather(map_v, [idx])
                res = (idx < _CELLS) & (val > cur)
                bad = bad | jnp.where(res, one16, zero16)
            acc_v[pl.ds(0, 16)] = acc_v[pl.ds(0, 16)] | bad
            return cc

        lax.fori_loop(0, _N // 128, detect_pass, 0)

        @pl.when(jnp.any(acc_v[pl.ds(0, 16)] > 0))
        def _sweep():
            def rnd(r, cc):
                def redo(g, cc2):
                    idx = lin_v[pl.ds(g * 16, 16)]
                    val = feats_v[pl.ds(g * 16, 16)]
                    cur = plsc.load_gather(map_v, [idx])
                    plsc.store_scatter(map_v, [idx], val, mask=val > cur)
                    return cc2

                return lax.fori_loop(0, _N // 16, redo, cc)

            lax.fori_loop(0, 16, rnd, 0)

        def fin_step(g, cc):
            for k in range(8):
                sl = pl.ds((g * 8 + k) * 16, 16)
                v = map_v[sl]
                map_v[sl] = jnp.where(v == _EPS, 0.0, v)
            return cc

        lax.fori_loop(0, _CELLS_PAD // 128, fin_step, 0)
        pltpu.sync_copy(map_v, out_hbm.at[row])
        return carry

    lax.fori_loop(0, _FPW, pair, 0)


def kernel(img_feats, depth):
    dsub = depth[:, 0, 2::4, 2::4]
    lin = pl.pallas_call(
        _index_body,
        out_shape=jax.ShapeDtypeStruct((_BS, 128, 128), jnp.int32),
    )(dsub)
    lin2 = lin.reshape(_BS, _N)
    feats2 = img_feats.reshape(_BS * _FC, _N)
    mesh = plsc.VectorSubcoreMesh(
        core_axis_name="c", subcore_axis_name="s",
        num_cores=_NC, num_subcores=_NS,
    )
    out = pl.kernel(
        _sc_scatter,
        out_type=jax.ShapeDtypeStruct((_BS * _FC, _CELLS_PAD), jnp.float32),
        mesh=mesh,
        scratch_types=[
            pltpu.VMEM((_N,), jnp.int32),
            pltpu.VMEM((_N,), jnp.float32),
            pltpu.VMEM((_CELLS_PAD,), jnp.float32),
            pltpu.VMEM((16,), jnp.int32),
        ],
        compiler_params=pltpu.CompilerParams(needs_layout_passes=False),
    )(feats2, lin2)
    return out[:, :_CELLS].reshape(_BS, _FC, _MAP_HW, _MAP_HW)


# speedup vs baseline: 10.4288x; 10.4288x over previous
import math

import jax
import jax.numpy as jnp
from jax import lax
from jax.experimental import pallas as pl
from jax.experimental.pallas import tpu as pltpu

_BS = 16
_FC = 128
_N = 16384
_MAP_HW = 101
_CELLS = _MAP_HW * _MAP_HW
_CELLS_PAD = 10208
_SENT = _CELLS
_EPS = -1e16
_MAP_SCALE = 0.1
_MAX_DEPTH = 10.0
_HFOV = math.radians(90.0)
_W = 512
_FX = _W / 2 * (1.0 / math.tan(_HFOV / 2))
_CX = _W / 2


def _index_body(dsub_ref, lin_ref):
    z = dsub_ref[...] * _MAX_DEPTH
    valid = jnp.abs(z) > 0.8
    zf = jnp.round(-(z / _MAP_SCALE) + (_MAP_HW - 1))
    j = lax.broadcasted_iota(jnp.int32, (_BS, 128, 128), 2).astype(jnp.float32)
    x = j * 4.0 + 2.0
    xx = (x - _CX) / _FX
    xf = jnp.round((xx * z) / _MAP_SCALE + (_MAP_HW - 1) / 2)
    r0 = zf.astype(jnp.int32)
    c0 = xf.astype(jnp.int32)
    invalid = (
        (r0 >= _MAP_HW) | (c0 >= _MAP_HW) | (r0 < 0) | (c0 < 0)
        | jnp.logical_not(valid)
    )
    lin_ref[...] = jnp.where(invalid, _SENT, r0 * _MAP_HW + c0)


def _scatter_body(lin_ref, feats_ref, out_ref):
    out_ref[...] = jnp.full(out_ref.shape, _EPS, jnp.float32)

    def step(p, carry):
        r = lin_ref[0, 0, p]
        row = feats_ref[0, pl.ds(p, 1), :]
        cur = out_ref[0, pl.ds(r, 1), :]
        out_ref[0, pl.ds(r, 1), :] = jnp.maximum(cur, row)
        return carry

    lax.fori_loop(0, _N, step, 0)
    cleaned = out_ref[...]
    out_ref[...] = jnp.where(cleaned == _EPS, 0.0, cleaned)


def kernel(img_feats, depth):
    dsub = depth[:, 0, 2::4, 2::4]
    lin = pl.pallas_call(
        _index_body,
        out_shape=jax.ShapeDtypeStruct((_BS, 128, 128), jnp.int32),
    )(dsub)
    lin3 = lin.reshape(_BS, 1, _N)
    feats_t = img_feats.reshape(_BS, _FC, _N).transpose(0, 2, 1)
    out = pl.pallas_call(
        _scatter_body,
        grid=(_BS,),
        in_specs=[
            pl.BlockSpec((1, 1, _N), lambda b: (b, 0, 0),
                         memory_space=pltpu.SMEM),
            pl.BlockSpec((1, _N, _FC), lambda b: (b, 0, 0)),
        ],
        out_specs=pl.BlockSpec((1, _CELLS_PAD, _FC), lambda b: (b, 0, 0)),
        out_shape=jax.ShapeDtypeStruct((_BS, _CELLS_PAD, _FC), jnp.float32),
    )(lin3, feats_t)
    out = out[:, :_CELLS, :].transpose(0, 2, 1)
    return out.reshape(_BS, _FC, _MAP_HW, _MAP_HW)
